# Initial kernel scaffold; baseline (speedup 1.0000x reference)
import jax
import jax.numpy as jnp
from jax.experimental import pallas as pl


def kernel(x, batch, W_ih, W_hh, b_ih, b_hh):
    raise NotImplementedError("write your pallas kernel here")


# speedup vs baseline: 11.1767x; 11.1767x over previous
import functools

import jax
import jax.numpy as jnp
from jax.experimental import pallas as pl
from jax.experimental.pallas import tpu as pltpu

_N = 100000
_C = 128
_B = 512
_T = 4
_BLK = 1024
_NB = (_N + _BLK - 1) // _BLK


def _body(x_ref, bat_ref, wih_ref, whh_ref, bias_ref, out_ref,
          qs_s, h_s, c_s, m_s, d_s, s_s):
    t = pl.program_id(0)
    nb = pl.program_id(1)
    neg = jnp.float32(-jnp.inf)

    @pl.when(jnp.logical_and(t == 0, nb == 0))
    def _init():
        qs_s[...] = jnp.zeros_like(qs_s)
        h_s[...] = jnp.zeros_like(h_s)
        c_s[...] = jnp.zeros_like(c_s)

    @pl.when(nb == 0)
    def _lstm():
        gates = (
            jax.lax.dot_general(qs_s[...], wih_ref[...], (((1,), (1,)), ((), ())),
                                preferred_element_type=jnp.float32)
            + jax.lax.dot_general(h_s[...], whh_ref[...], (((1,), (1,)), ((), ())),
                                  preferred_element_type=jnp.float32)
            + bias_ref[...]
        )
        i_g = jax.nn.sigmoid(gates[:, 0 * _C:1 * _C])
        f_g = jax.nn.sigmoid(gates[:, 1 * _C:2 * _C])
        g_g = jnp.tanh(gates[:, 2 * _C:3 * _C])
        o_g = jax.nn.sigmoid(gates[:, 3 * _C:4 * _C])
        c = f_g * c_s[...] + i_g * g_g
        h_s[...] = o_g * jnp.tanh(c)
        c_s[...] = c
        m_s[...] = jnp.full_like(m_s, neg)
        d_s[...] = jnp.zeros_like(d_s)
        s_s[...] = jnp.zeros_like(s_s)

    row = jax.lax.broadcasted_iota(jnp.int32, (_BLK, _C), 0)
    valid = (nb * _BLK + row) < _N
    xb = jnp.where(valid, x_ref[...], 0.0)
    bat = bat_ref[0]

    seg = jax.lax.broadcasted_iota(jnp.int32, (_B, _BLK), 0)
    onehot = seg == bat

    xq = jax.lax.dot_general(h_s[...], xb, (((1,), (1,)), ((), ())),
                             preferred_element_type=jnp.float32)
    masked = jnp.where(onehot, xq, neg)
    m_old = m_s[...]
    m_new = jnp.maximum(m_old, jnp.max(masked, axis=1, keepdims=True))
    scale = jnp.where(m_old > neg, jnp.exp(m_old - m_new), 0.0)

    diff = jnp.sum(jnp.where(onehot, xq - m_new, 0.0), axis=0, keepdims=True)
    w_node = jnp.exp(diff)
    wmat = jnp.where(onehot, w_node, 0.0)

    d_s[...] = d_s[...] * scale + jnp.sum(wmat, axis=1, keepdims=True)
    s_s[...] = s_s[...] * scale + jax.lax.dot_general(
        wmat, xb, (((1,), (0,)), ((), ())), preferred_element_type=jnp.float32)
    m_s[...] = m_new

    @pl.when(nb == _NB - 1)
    def _finalize():
        r = s_s[...] / (d_s[...] + 1e-16)
        qs_s[...] = jnp.concatenate([h_s[...], r], axis=1)

    @pl.when(jnp.logical_and(t == _T - 1, nb == _NB - 1))
    def _emit():
        out_ref[...] = qs_s[...]


@jax.jit
def kernel(x, batch, W_ih, W_hh, b_ih, b_hh):
    npad = _NB * _BLK
    bat = jnp.concatenate(
        [batch.astype(jnp.int32), jnp.full((npad - _N,), _B, jnp.int32)]
    ).reshape(_NB, 1, _BLK)
    bias = (b_ih + b_hh).reshape(1, 4 * _C)

    return pl.pallas_call(
        _body,
        grid=(_T, _NB),
        in_specs=[
            pl.BlockSpec((_BLK, _C), lambda t, nb: (nb, 0)),
            pl.BlockSpec((1, 1, _BLK), lambda t, nb: (nb, 0, 0)),
            pl.BlockSpec((4 * _C, 2 * _C), lambda t, nb: (0, 0)),
            pl.BlockSpec((4 * _C, _C), lambda t, nb: (0, 0)),
            pl.BlockSpec((1, 4 * _C), lambda t, nb: (0, 0)),
        ],
        out_specs=pl.BlockSpec((_B, 2 * _C), lambda t, nb: (0, 0)),
        out_shape=jax.ShapeDtypeStruct((_B, 2 * _C), jnp.float32),
        scratch_shapes=[
            pltpu.VMEM((_B, 2 * _C), jnp.float32),
            pltpu.VMEM((_B, _C), jnp.float32),
            pltpu.VMEM((_B, _C), jnp.float32),
            pltpu.VMEM((_B, 1), jnp.float32),
            pltpu.VMEM((_B, 1), jnp.float32),
            pltpu.VMEM((_B, _C), jnp.float32),
        ],
        compiler_params=pltpu.CompilerParams(
            dimension_semantics=("arbitrary", "arbitrary"),
        ),
    )(x, bat, W_ih, W_hh, bias)
